# baseline (speedup 1.0000x reference)
import jax
import jax.numpy as jnp
from jax import lax
from jax.experimental import pallas as pl
from jax.experimental.pallas import tpu as pltpu
from jax.experimental.pallas import tpu_sc as plsc

NUM_CLASSES = 100000
HD = 128
N = 16384

NC = 2
NS = 16

BLOCK = 12800
NB = 4
DUMMY = BLOCK
ACC_ROWS = BLOCK + 8
LPT = N // NS
LIST_CAP = LPT + 128
RPT = BLOCK // NS
CH = 64
MAXCH = LPT // CH


def _body(labels_hbm, hv_hbm, w_hbm, out_hbm,
          labels_v, stage, src_list, dst_list, srcidx, dstidx, acc,
          sem_i, sem_o, semg0, semg1):
    c = lax.axis_index("c")
    s = lax.axis_index("s")
    lab_base = s * LPT
    iota16 = lax.iota(jnp.int32, 16)
    dummy16 = jnp.full((16,), DUMMY, jnp.int32)

    pltpu.sync_copy(labels_hbm.at[pl.ds(lab_base, LPT)], labels_v)

    def blo_of(b):
        return (c * NB + b) * BLOCK

    def fire_init(b):
        row0 = blo_of(b) + s * RPT

        @pl.when(row0 < NUM_CLASSES)
        def _():
            pltpu.async_copy(
                w_hbm.at[pl.ds(row0, RPT)], acc.at[pl.ds(s * RPT, RPT)], sem_i
            )

    def wait_init(b):
        row0 = blo_of(b) + s * RPT

        @pl.when(row0 < NUM_CLASSES)
        def _():
            pltpu.make_async_copy(
                w_hbm.at[pl.ds(row0, RPT)], acc.at[pl.ds(s * RPT, RPT)], sem_i
            ).wait()

    def do_scan(b):
        blo = blo_of(b)
        bhi = blo + BLOCK

        def _prefill(i, carry):
            src_list[pl.ds(i * 16, 16)] = dummy16
            dst_list[pl.ds(i * 16, 16)] = dummy16
            return carry

        lax.fori_loop(0, LPT // 16, _prefill, 0)

        def _scan(j, cnt):
            lab = labels_v[pl.ds(j * 16, 16)]
            m = (lab >= blo) & (lab < bhi)
            plsc.store_compressed(dst_list.at[pl.ds(cnt, 16)], lab - blo, mask=m)
            plsc.store_compressed(
                src_list.at[pl.ds(cnt, 16)], lab_base + j * 16 + iota16, mask=m
            )
            return cnt + jnp.sum(m.astype(jnp.int32))

        cnt = lax.fori_loop(0, LPT // 16, _scan, 0)
        return (cnt + CH - 1) >> 6

    def load_srcidx(k, par):
        base = k * CH
        for g in range(CH // 16):
            srcidx[par, pl.ds(g * 16, 16)] = src_list[pl.ds(base + g * 16, 16)]

    def fire_gather(k, par, sem):
        load_srcidx(k, par)
        pltpu.async_copy(hv_hbm.at[srcidx.at[par]], stage.at[par], sem)

    def wait_gather(k, par, sem):
        pltpu.make_async_copy(
            hv_hbm.at[srcidx.at[par]], stage.at[par], sem
        ).wait()

    fire_init(0)
    nch = do_scan(0)

    for b in range(NB):
        wait_init(b)
        plsc.subcore_barrier()

        @pl.when(0 < nch)
        def _():
            fire_gather(0, 0, semg0)

        for k in range(MAXCH):
            par = k % 2
            sem = semg0 if par == 0 else semg1
            npar = (k + 1) % 2
            nsem = semg0 if npar == 0 else semg1

            @pl.when(k + 1 < nch)
            def _():
                fire_gather(k + 1, npar, nsem)

            @pl.when(k < nch)
            def _():
                base = k * CH
                for g in range(CH // 16):
                    dstidx[pl.ds(g * 16, 16)] = dst_list[pl.ds(base + g * 16, 16)]
                wait_gather(k, par, sem)
                pltpu.sync_copy(stage.at[par], acc.at[dstidx], add=True)

        plsc.subcore_barrier()

        row0 = blo_of(b) + s * RPT

        @pl.when(row0 < NUM_CLASSES)
        def _():
            pltpu.async_copy(
                acc.at[pl.ds(s * RPT, RPT)], out_hbm.at[pl.ds(row0, RPT)], sem_o
            )

        if b + 1 < NB:
            nch = do_scan(b + 1)

        @pl.when(row0 < NUM_CLASSES)
        def _():
            pltpu.make_async_copy(
                acc.at[pl.ds(s * RPT, RPT)], out_hbm.at[pl.ds(row0, RPT)], sem_o
            ).wait()

        if b + 1 < NB:
            fire_init(b + 1)


@jax.jit
def _scatter_add(labels, hv, classify_weights):
    mesh = plsc.VectorSubcoreMesh(
        core_axis_name="c", subcore_axis_name="s", num_cores=NC, num_subcores=NS
    )
    return pl.kernel(
        _body,
        out_type=jax.ShapeDtypeStruct((NUM_CLASSES, HD), jnp.float32),
        mesh=mesh,
        compiler_params=pltpu.CompilerParams(needs_layout_passes=False),
        scratch_types=[
            pltpu.VMEM((LPT,), jnp.int32),
            pltpu.VMEM((2, CH, HD), jnp.float32),
            pltpu.VMEM((LIST_CAP,), jnp.int32),
            pltpu.VMEM((LIST_CAP,), jnp.int32),
            pltpu.VMEM((2, CH), jnp.int32),
            pltpu.VMEM((CH,), jnp.int32),
            pltpu.VMEM_SHARED((ACC_ROWS, HD), jnp.float32),
            pltpu.SemaphoreType.DMA,
            pltpu.SemaphoreType.DMA,
            pltpu.SemaphoreType.DMA,
            pltpu.SemaphoreType.DMA,
        ],
    )(labels, hv, classify_weights)


def kernel(labels, hv, classify_weights):
    return _scatter_add(labels, hv, classify_weights)


# speedup vs baseline: 1.6386x; 1.6386x over previous
import jax
import jax.numpy as jnp
from jax import lax
from jax.experimental import pallas as pl
from jax.experimental.pallas import tpu as pltpu
from jax.experimental.pallas import tpu_sc as plsc

NUM_CLASSES = 100000
HD = 128
N = 16384

NC = 2
NS = 16

BLOCK = 12800
NB = 4
DUMMY = BLOCK
ACC_ROWS = BLOCK + 8
LPT = N // NS
RPT = BLOCK // NS
CH = 64
NCH = LPT // CH


def _body(labels_hbm, hv_hbm, w_hbm, out_hbm,
          labels_v, stage, dstidx, acc, sem_i, sem_o, semg0, semg1):
    c = lax.axis_index("c")
    s = lax.axis_index("s")
    lab_base = s * LPT

    pltpu.sync_copy(labels_hbm.at[pl.ds(lab_base, LPT)], labels_v)

    def blo_of(b):
        return (c * NB + b) * BLOCK

    def fire_init(b):
        row0 = blo_of(b) + s * RPT

        @pl.when(row0 < NUM_CLASSES)
        def _():
            pltpu.async_copy(
                w_hbm.at[pl.ds(row0, RPT)], acc.at[pl.ds(s * RPT, RPT)], sem_i
            )

    def wait_init(b):
        row0 = blo_of(b) + s * RPT

        @pl.when(row0 < NUM_CLASSES)
        def _():
            pltpu.make_async_copy(
                w_hbm.at[pl.ds(row0, RPT)], acc.at[pl.ds(s * RPT, RPT)], sem_i
            ).wait()

    def fire_load(k, par, sem):
        pltpu.async_copy(
            hv_hbm.at[pl.ds(lab_base + k * CH, CH)], stage.at[par], sem
        )

    def wait_load(k, par, sem):
        pltpu.make_async_copy(
            hv_hbm.at[pl.ds(lab_base + k * CH, CH)], stage.at[par], sem
        ).wait()

    fire_init(0)

    for b in range(NB):
        blo = blo_of(b)
        bhi = blo + BLOCK

        wait_init(b)
        plsc.subcore_barrier()

        fire_load(0, 0, semg0)
        for k in range(NCH):
            par = k % 2
            sem = semg0 if par == 0 else semg1
            npar = (k + 1) % 2
            nsem = semg0 if npar == 0 else semg1
            if k + 1 < NCH:
                fire_load(k + 1, npar, nsem)
            for g in range(CH // 16):
                lab = labels_v[pl.ds(k * CH + g * 16, 16)]
                in_blk = (lab >= blo) & (lab < bhi)
                dstidx[pl.ds(g * 16, 16)] = jnp.where(in_blk, lab - blo, DUMMY)
            wait_load(k, par, sem)
            pltpu.sync_copy(stage.at[par], acc.at[dstidx], add=True)

        plsc.subcore_barrier()

        row0 = blo + s * RPT

        @pl.when(row0 < NUM_CLASSES)
        def _():
            pltpu.async_copy(
                acc.at[pl.ds(s * RPT, RPT)], out_hbm.at[pl.ds(row0, RPT)], sem_o
            )

        @pl.when(row0 < NUM_CLASSES)
        def _():
            pltpu.make_async_copy(
                acc.at[pl.ds(s * RPT, RPT)], out_hbm.at[pl.ds(row0, RPT)], sem_o
            ).wait()

        if b + 1 < NB:
            fire_init(b + 1)


@jax.jit
def _scatter_add(labels, hv, classify_weights):
    mesh = plsc.VectorSubcoreMesh(
        core_axis_name="c", subcore_axis_name="s", num_cores=NC, num_subcores=NS
    )
    return pl.kernel(
        _body,
        out_type=jax.ShapeDtypeStruct((NUM_CLASSES, HD), jnp.float32),
        mesh=mesh,
        compiler_params=pltpu.CompilerParams(needs_layout_passes=False),
        scratch_types=[
            pltpu.VMEM((LPT,), jnp.int32),
            pltpu.VMEM((2, CH, HD), jnp.float32),
            pltpu.VMEM((CH,), jnp.int32),
            pltpu.VMEM_SHARED((ACC_ROWS, HD), jnp.float32),
            pltpu.SemaphoreType.DMA,
            pltpu.SemaphoreType.DMA,
            pltpu.SemaphoreType.DMA,
            pltpu.SemaphoreType.DMA,
        ],
    )(labels, hv, classify_weights)


def kernel(labels, hv, classify_weights):
    return _scatter_add(labels, hv, classify_weights)
